# baseline (speedup 1.0000x reference)
import functools

import jax
import jax.numpy as jnp
from jax import lax
from jax.experimental import pallas as pl
from jax.experimental.pallas import tpu as pltpu
from jax.experimental.pallas import tpu_sc as plsc

NC = 2
NS = 16
NW = NC * NS

C = 512
LANES = 16
GCHUNK = 128


def _gather_tasks(ids_t, tbl2):
    F, S = ids_t.shape
    D = 32
    n_tasks = F * (S // C)
    tasks_per_w = n_tasks // NW
    chunks_per_row = S // C

    mesh = plsc.VectorSubcoreMesh(core_axis_name="c", subcore_axis_name="s")

    @functools.partial(
        pl.kernel,
        mesh=mesh,
        out_type=jax.ShapeDtypeStruct((F, D, S), jnp.float32),
        scratch_types=[
            pltpu.VMEM((C,), jnp.int32),
            pltpu.VMEM((C,), jnp.int32),
            pltpu.VMEM((C,), jnp.int32),
            pltpu.VMEM((C, 128), jnp.float32),
            pltpu.VMEM((D, C), jnp.float32),
            pltpu.SemaphoreType.DMA,
        ],
        compiler_params=pltpu.CompilerParams(needs_layout_passes=False),
    )
    def k(ids_hbm, tbl_hbm, out_hbm, idr_v, idq_v, off_v, rows_v, out_v, sem):
        wid = lax.axis_index("s") * NC + lax.axis_index("c")

        def task(kk, _):
            t = wid * tasks_per_w + kk
            f = t // chunks_per_row
            s0 = (t % chunks_per_row) * C
            pltpu.sync_copy(ids_hbm.at[f, pl.ds(s0, C)], idr_v)
            for i in range(C // LANES):
                v = idr_v[pl.ds(i * LANES, LANES)]
                idq_v[pl.ds(i * LANES, LANES)] = lax.shift_right_logical(v, 2)
                off_v[pl.ds(i * LANES, LANES)] = lax.shift_left(
                    lax.bitwise_and(v, 3), 5)
            copies = [
                pltpu.async_copy(
                    tbl_hbm.at[idq_v.at[pl.ds(j * GCHUNK, GCHUNK)]],
                    rows_v.at[pl.ds(j * GCHUNK, GCHUNK)],
                    sem,
                )
                for j in range(C // GCHUNK)
            ]
            for cp in copies:
                cp.wait()

            def col(d, _):
                for j0 in range(0, C, LANES):
                    rowi = lax.iota(jnp.int32, LANES) + j0
                    coli = off_v[pl.ds(j0, LANES)] + d
                    out_v[d, pl.ds(j0, LANES)] = plsc.load_gather(
                        rows_v, [rowi, coli])
                return ()

            lax.fori_loop(0, D, col, (), unroll=False)
            pltpu.sync_copy(out_v, out_hbm.at[f, :, pl.ds(s0, C)])
            return ()

        lax.fori_loop(0, tasks_per_w, task, (), unroll=False)

    return k(ids_t, tbl2)


def kernel(input_ids, table):
    S, F = input_ids.shape
    D = table.shape[1]
    ids_t = input_ids.T.astype(jnp.int32)
    tbl2 = table.reshape(-1, 128)
    out3 = _gather_tasks(ids_t, tbl2)
    return jnp.transpose(out3, (2, 0, 1))
